# baseline (speedup 1.0000x reference)
import jax
import jax.numpy as jnp
from jax import lax
from jax.experimental import pallas as pl
from jax.experimental.pallas import tpu as pltpu
from jax.experimental.pallas import tpu_sc as plsc

_B, _L, _DIM = 1024, 512, 128
_N = _B * _L

_info = plsc.get_sparse_core_info()
_NC = _info.num_cores
_NS = _info.num_subcores
_NW = _NC * _NS
_ROWS_PER_W = _N // _NW
_C = 128
_CHUNKS = _ROWS_PER_W // _C


def _embed_body(tok_hbm, seg_hbm, pos_hbm, ttab, stab, ptab, out_hbm,
                idx_t, idx_s, idx_p, rows_t, rows_s, rows_p, sem):
    wid = lax.axis_index("s") * _NC + lax.axis_index("c")
    base_w = wid * _ROWS_PER_W

    def chunk(g, carry):
        base = base_w + g * _C
        pltpu.sync_copy(tok_hbm.at[pl.ds(base, _C)], idx_t)
        pltpu.sync_copy(seg_hbm.at[pl.ds(base, _C)], idx_s)
        pltpu.sync_copy(pos_hbm.at[pl.ds(base, _C)], idx_p)
        c1 = pltpu.async_copy(ttab.at[idx_t], rows_t, sem)
        c2 = pltpu.async_copy(stab.at[idx_s], rows_s, sem)
        c3 = pltpu.async_copy(ptab.at[idx_p], rows_p, sem)
        c1.wait()
        c2.wait()
        c3.wait()

        def row(r, carry2):
            for j in range(_DIM // 16):
                sl = pl.ds(j * 16, 16)
                rows_t[r, sl] = rows_t[r, sl] + rows_s[r, sl] + rows_p[r, sl]
            return carry2

        lax.fori_loop(0, _C, row, 0)
        pltpu.sync_copy(rows_t, out_hbm.at[pl.ds(base, _C)])
        return carry

    lax.fori_loop(0, _CHUNKS, chunk, 0)


def kernel(tokens, segment_ids, pos_ids, token_table, segment_table, pos_table):
    tok = jnp.reshape(tokens, (_N,)).astype(jnp.int32)
    seg = jnp.reshape(segment_ids, (_N,)).astype(jnp.int32)
    pos = jnp.reshape(pos_ids, (_N,)).astype(jnp.int32)
    mesh = plsc.VectorSubcoreMesh(core_axis_name="c", subcore_axis_name="s")
    run = pl.kernel(
        _embed_body,
        mesh=mesh,
        out_type=jax.ShapeDtypeStruct((_N, _DIM), jnp.float32),
        scratch_types=[
            pltpu.VMEM((_C,), jnp.int32),
            pltpu.VMEM((_C,), jnp.int32),
            pltpu.VMEM((_C,), jnp.int32),
            pltpu.VMEM((_C, _DIM), jnp.float32),
            pltpu.VMEM((_C, _DIM), jnp.float32),
            pltpu.VMEM((_C, _DIM), jnp.float32),
            pltpu.SemaphoreType.DMA,
        ],
    )
    out = run(tok, seg, pos, token_table, segment_table, pos_table)
    return jnp.reshape(out, (_B, _L, _DIM))


# speedup vs baseline: 27.8236x; 27.8236x over previous
import jax
import jax.numpy as jnp
from jax import lax
from jax.experimental import pallas as pl
from jax.experimental.pallas import tpu as pltpu
from jax.experimental.pallas import tpu_sc as plsc

_B, _L, _DIM = 1024, 512, 128
_N = _B * _L
_VEC = 16
_NVJ = _DIM // _VEC

_info = plsc.get_sparse_core_info()
_NC = _info.num_cores
_NS = _info.num_subcores
_NW = _NC * _NS
_ROWS_PER_W = _N // _NW
_C = 128
_KB = 16
_BLK = _C * _KB
_NSC = _ROWS_PER_W // _BLK

_MAX_LEN = 512
_COMB_ROWS = 2 * _MAX_LEN
_CB_PER_W = _COMB_ROWS // _NW


def _comb_body(stab_hbm, ptab_hbm, comb_hbm, srows, prows):
    wid = lax.axis_index("s") * _NC + lax.axis_index("c")
    base = pl.multiple_of(wid * _CB_PER_W, _CB_PER_W)
    s = base // _MAX_LEN
    p0 = pl.multiple_of(lax.rem(base, _MAX_LEN), _CB_PER_W)
    pltpu.sync_copy(stab_hbm, srows)
    pltpu.sync_copy(ptab_hbm.at[pl.ds(p0, _CB_PER_W)], prows)

    def row(r, carry):
        for j in range(_NVJ):
            sl = pl.ds(j * _VEC, _VEC)
            prows[r, sl] = prows[r, sl] + srows[s, sl]
        return carry

    lax.fori_loop(0, _CB_PER_W, row, 0)
    pltpu.sync_copy(prows, comb_hbm.at[pl.ds(base, _CB_PER_W)])


def _embed_body(tok_hbm, seg_hbm, pos_hbm, ttab, comb, out_hbm,
                idxt, idxs, idxp, idxc,
                rt0, rt1, rc0, rc1, st0, st1, sc0, sc1):
    wid = lax.axis_index("s") * _NC + lax.axis_index("c")
    base_w = wid * _ROWS_PER_W

    def fire(k, rt, rc, semt, semc):
        pltpu.async_copy(ttab.at[idxt.at[k]], rt, semt)
        pltpu.async_copy(comb.at[idxc.at[k]], rc, semc)

    def drain(rt, rc, semt, semc):
        pltpu.make_async_copy(ttab.at[idxt.at[0]], rt, semt).wait()
        pltpu.make_async_copy(comb.at[idxc.at[0]], rc, semc).wait()

    def superchunk(g, carry):
        row0 = pl.multiple_of(base_w + g * _BLK, _BLK)
        blk = pl.multiple_of(row0 // _C, _KB)
        pltpu.sync_copy(tok_hbm.at[pl.ds(blk, _KB)], idxt)
        pltpu.sync_copy(seg_hbm.at[pl.ds(blk, _KB)], idxs)
        pltpu.sync_copy(pos_hbm.at[pl.ds(blk, _KB)], idxp)

        def crow(k, carry2):
            for j in range(_C // _VEC):
                sl = pl.ds(j * _VEC, _VEC)
                idxc[k, sl] = idxs[k, sl] * _MAX_LEN + idxp[k, sl]
            return carry2

        lax.fori_loop(0, _KB, crow, 0)

        bufs = ((rt0, rc0, st0, sc0), (rt1, rc1, st1, sc1))
        fire(0, *bufs[0])

        def pair(kk, carry2):
            for b in range(2):
                k = kk * 2 + b
                rt, rc, semt, semc = bufs[b]

                @pl.when(k < _KB - 1)
                def _():
                    fire(k + 1, *bufs[1 - b])

                drain(rt, rc, semt, semc)

                def addrow(r, carry3):
                    for j in range(_NVJ):
                        sl = pl.ds(j * _VEC, _VEC)
                        rt[r, sl] = rt[r, sl] + rc[r, sl]
                    return carry3

                lax.fori_loop(0, _C, addrow, 0)
                pltpu.sync_copy(
                    rt, out_hbm.at[pl.ds(pl.multiple_of(row0 + k * _C, _C), _C)])
            return carry2

        lax.fori_loop(0, _KB // 2, pair, 0)
        return carry

    lax.fori_loop(0, _NSC, superchunk, 0)


def kernel(tokens, segment_ids, pos_ids, token_table, segment_table, pos_table):
    tok = jnp.reshape(tokens, (_N // _C, _C)).astype(jnp.int32)
    seg = jnp.reshape(segment_ids, (_N // _C, _C)).astype(jnp.int32)
    pos = jnp.reshape(pos_ids, (_N // _C, _C)).astype(jnp.int32)
    mesh = plsc.VectorSubcoreMesh(core_axis_name="c", subcore_axis_name="s")

    comb = pl.kernel(
        _comb_body,
        mesh=mesh,
        out_type=jax.ShapeDtypeStruct((_COMB_ROWS, _DIM), jnp.float32),
        scratch_types=[
            pltpu.VMEM((2, _DIM), jnp.float32),
            pltpu.VMEM((_CB_PER_W, _DIM), jnp.float32),
        ],
    )(segment_table, pos_table)

    out = pl.kernel(
        _embed_body,
        mesh=mesh,
        out_type=jax.ShapeDtypeStruct((_N, _DIM), jnp.float32),
        scratch_types=[
            pltpu.VMEM((_KB, _C), jnp.int32),
            pltpu.VMEM((_KB, _C), jnp.int32),
            pltpu.VMEM((_KB, _C), jnp.int32),
            pltpu.VMEM((_KB, _C), jnp.int32),
            pltpu.VMEM((_C, _DIM), jnp.float32),
            pltpu.VMEM((_C, _DIM), jnp.float32),
            pltpu.VMEM((_C, _DIM), jnp.float32),
            pltpu.VMEM((_C, _DIM), jnp.float32),
            pltpu.SemaphoreType.DMA,
            pltpu.SemaphoreType.DMA,
            pltpu.SemaphoreType.DMA,
            pltpu.SemaphoreType.DMA,
        ],
    )(tok, seg, pos, token_table, comb)
    return jnp.reshape(out, (_B, _L, _DIM))


# speedup vs baseline: 28.5647x; 1.0266x over previous
import jax
import jax.numpy as jnp
from jax import lax
from jax.experimental import pallas as pl
from jax.experimental.pallas import tpu as pltpu
from jax.experimental.pallas import tpu_sc as plsc

_B, _L, _DIM = 1024, 512, 128
_N = _B * _L
_VEC = 16
_NVJ = _DIM // _VEC

_info = plsc.get_sparse_core_info()
_NC = _info.num_cores
_NS = _info.num_subcores
_NW = _NC * _NS
_ROWS_PER_W = _N // _NW
_C = 128
_KB = 16
_BLK = _C * _KB
_NSC = _ROWS_PER_W // _BLK

_MAX_LEN = 512
_COMB_ROWS = 2 * _MAX_LEN
_CB_PER_W = _COMB_ROWS // _NW


def _comb_body(stab_hbm, ptab_hbm, comb_hbm, srows, prows):
    wid = lax.axis_index("s") * _NC + lax.axis_index("c")
    base = pl.multiple_of(wid * _CB_PER_W, _CB_PER_W)
    s = base // _MAX_LEN
    p0 = pl.multiple_of(lax.rem(base, _MAX_LEN), _CB_PER_W)
    pltpu.sync_copy(stab_hbm, srows)
    pltpu.sync_copy(ptab_hbm.at[pl.ds(p0, _CB_PER_W)], prows)

    def row(r, carry):
        for j in range(_NVJ):
            sl = pl.ds(j * _VEC, _VEC)
            prows[r, sl] = prows[r, sl] + srows[s, sl]
        return carry

    lax.fori_loop(0, _CB_PER_W, row, 0)
    pltpu.sync_copy(prows, comb_hbm.at[pl.ds(base, _CB_PER_W)])


def _embed_body(tok_hbm, seg_hbm, pos_hbm, ttab, comb, out_hbm,
                idxt, idxs, idxp, idxc,
                rt0, rt1, rc0, rc1, st0, st1, sc0, sc1, sw0, sw1):
    wid = lax.axis_index("s") * _NC + lax.axis_index("c")
    base_w = wid * _ROWS_PER_W
    n_chunks = _ROWS_PER_W // _C
    bufs = ((rt0, rc0, st0, sc0, sw0), (rt1, rc1, st1, sc1, sw1))

    def load_idx_block(k):
        blk = pl.multiple_of((base_w + k * _C) // _C, _KB)
        pltpu.sync_copy(tok_hbm.at[pl.ds(blk, _KB)], idxt)
        pltpu.sync_copy(seg_hbm.at[pl.ds(blk, _KB)], idxs)
        pltpu.sync_copy(pos_hbm.at[pl.ds(blk, _KB)], idxp)

        def crow(r, carry):
            for j in range(_C // _VEC):
                sl = pl.ds(j * _VEC, _VEC)
                idxc[r, sl] = idxs[r, sl] * _MAX_LEN + idxp[r, sl]
            return carry

        lax.fori_loop(0, _KB, crow, 0)

    def fire(k, rt, rc, semt, semc):
        kb = lax.rem(k, _KB)
        pltpu.async_copy(ttab.at[idxt.at[kb]], rt, semt)
        pltpu.async_copy(comb.at[idxc.at[kb]], rc, semc)

    def drain_gathers(rt, rc, semt, semc):
        pltpu.make_async_copy(ttab.at[idxt.at[0]], rt, semt).wait()
        pltpu.make_async_copy(comb.at[idxc.at[0]], rc, semc).wait()

    def drain_write(rt, semw):
        pltpu.make_async_copy(rt, out_hbm.at[pl.ds(0, _C)], semw).wait()

    load_idx_block(0)
    fire(0, *bufs[0][:4])

    def pair(kk, carry):
        for b in range(2):
            k = kk * 2 + b
            rt, rc, semt, semc, semw = bufs[b]
            nrt, nrc, nsemt, nsemc, nsemw = bufs[1 - b]

            drain_gathers(rt, rc, semt, semc)

            if b == 1:
                @pl.when(k < n_chunks - 1)
                def _():
                    @pl.when(lax.rem(k + 1, _KB) == 0)
                    def _():
                        load_idx_block(k + 1)

                    drain_write(nrt, nsemw)
                    fire(k + 1, nrt, nrc, nsemt, nsemc)
            else:
                @pl.when(k >= 1)
                def _():
                    drain_write(nrt, nsemw)

                fire(k + 1, nrt, nrc, nsemt, nsemc)

            def addrow(r, carry2):
                for j in range(_NVJ):
                    sl = pl.ds(j * _VEC, _VEC)
                    plsc.addupdate(rt.at[r, sl], rc[r, sl])
                return carry2

            lax.fori_loop(0, _C, addrow, 0)
            pltpu.async_copy(
                rt,
                out_hbm.at[pl.ds(pl.multiple_of(base_w + k * _C, _C), _C)],
                semw)
        return carry

    lax.fori_loop(0, n_chunks // 2, pair, 0)

    drain_write(rt0, sw0)
    drain_write(rt1, sw1)


def kernel(tokens, segment_ids, pos_ids, token_table, segment_table, pos_table):
    tok = jnp.reshape(tokens, (_N // _C, _C)).astype(jnp.int32)
    seg = jnp.reshape(segment_ids, (_N // _C, _C)).astype(jnp.int32)
    pos = jnp.reshape(pos_ids, (_N // _C, _C)).astype(jnp.int32)
    mesh = plsc.VectorSubcoreMesh(core_axis_name="c", subcore_axis_name="s")

    comb = pl.kernel(
        _comb_body,
        mesh=mesh,
        out_type=jax.ShapeDtypeStruct((_COMB_ROWS, _DIM), jnp.float32),
        scratch_types=[
            pltpu.VMEM((2, _DIM), jnp.float32),
            pltpu.VMEM((_CB_PER_W, _DIM), jnp.float32),
        ],
    )(segment_table, pos_table)

    out = pl.kernel(
        _embed_body,
        mesh=mesh,
        out_type=jax.ShapeDtypeStruct((_N, _DIM), jnp.float32),
        scratch_types=[
            pltpu.VMEM((_KB, _C), jnp.int32),
            pltpu.VMEM((_KB, _C), jnp.int32),
            pltpu.VMEM((_KB, _C), jnp.int32),
            pltpu.VMEM((_KB, _C), jnp.int32),
            pltpu.VMEM((_C, _DIM), jnp.float32),
            pltpu.VMEM((_C, _DIM), jnp.float32),
            pltpu.VMEM((_C, _DIM), jnp.float32),
            pltpu.VMEM((_C, _DIM), jnp.float32),
            pltpu.SemaphoreType.DMA,
            pltpu.SemaphoreType.DMA,
            pltpu.SemaphoreType.DMA,
            pltpu.SemaphoreType.DMA,
            pltpu.SemaphoreType.DMA,
            pltpu.SemaphoreType.DMA,
        ],
    )(tok, seg, pos, token_table, comb)
    return jnp.reshape(out, (_B, _L, _DIM))


# speedup vs baseline: 38.3295x; 1.3418x over previous
import jax
import jax.numpy as jnp
from jax import lax
from jax.experimental import pallas as pl
from jax.experimental.pallas import tpu as pltpu
from jax.experimental.pallas import tpu_sc as plsc

_B, _L, _DIM = 1024, 512, 128
_N = _B * _L
_VEC = 16
_NVJ = _DIM // _VEC

_info = plsc.get_sparse_core_info()
_NC = _info.num_cores
_NS = _info.num_subcores
_NW = _NC * _NS
_ROWS_PER_W = _N // _NW
_C = 128
_KB = 16
_BLK = _C * _KB

_MAX_LEN = 512
_COMB_ROWS = 2 * _MAX_LEN
_CB_PER_T = _COMB_ROWS // _NS


def _embed_body(tok_hbm, seg_hbm, pos_hbm, ttab, stab, ptab, out_hbm,
                idxt, idxs, idxp, idxc,
                rt0, rt1, rc0, rc1, srows, prows, comb_sp,
                st0, st1, sc0, sc1, sw0, sw1):
    sid = lax.axis_index("s")
    wid = sid * _NC + lax.axis_index("c")
    base_w = wid * _ROWS_PER_W
    n_chunks = _ROWS_PER_W // _C
    bufs = ((rt0, rc0, st0, sc0, sw0), (rt1, rc1, st1, sc1, sw1))

    cb0 = pl.multiple_of(sid * _CB_PER_T, _CB_PER_T)
    s_of_tile = cb0 // _MAX_LEN
    p0 = pl.multiple_of(lax.rem(cb0, _MAX_LEN), _CB_PER_T)
    pltpu.sync_copy(stab, srows)
    pltpu.sync_copy(ptab.at[pl.ds(p0, _CB_PER_T)], prows)

    def brow(r, carry):
        for j in range(_NVJ):
            sl = pl.ds(j * _VEC, _VEC)
            prows[r, sl] = prows[r, sl] + srows[s_of_tile, sl]
        return carry

    lax.fori_loop(0, _CB_PER_T, brow, 0)
    pltpu.sync_copy(prows, comb_sp.at[pl.ds(cb0, _CB_PER_T)])
    plsc.subcore_barrier()

    def load_idx_block(k):
        blk = pl.multiple_of((base_w + k * _C) // _C, _KB)
        pltpu.sync_copy(tok_hbm.at[pl.ds(blk, _KB)], idxt)
        pltpu.sync_copy(seg_hbm.at[pl.ds(blk, _KB)], idxs)
        pltpu.sync_copy(pos_hbm.at[pl.ds(blk, _KB)], idxp)

        def crow(r, carry):
            for j in range(_C // _VEC):
                sl = pl.ds(j * _VEC, _VEC)
                idxc[r, sl] = idxs[r, sl] * _MAX_LEN + idxp[r, sl]
            return carry

        lax.fori_loop(0, _KB, crow, 0)

    def fire(k, rt, rc, semt, semc):
        kb = lax.rem(k, _KB)
        pltpu.async_copy(ttab.at[idxt.at[kb]], rt, semt)
        pltpu.async_copy(comb_sp.at[idxc.at[kb]], rc, semc)

    def drain_gathers(rt, rc, semt, semc):
        pltpu.make_async_copy(ttab.at[idxt.at[0]], rt, semt).wait()
        pltpu.make_async_copy(comb_sp.at[idxc.at[0]], rc, semc).wait()

    def drain_write(rt, semw):
        pltpu.make_async_copy(rt, out_hbm.at[pl.ds(0, _C)], semw).wait()

    load_idx_block(0)
    fire(0, *bufs[0][:4])

    def pair(kk, carry):
        for b in range(2):
            k = kk * 2 + b
            rt, rc, semt, semc, semw = bufs[b]
            nrt, nrc, nsemt, nsemc, nsemw = bufs[1 - b]

            drain_gathers(rt, rc, semt, semc)

            if b == 1:
                @pl.when(k < n_chunks - 1)
                def _():
                    @pl.when(lax.rem(k + 1, _KB) == 0)
                    def _():
                        load_idx_block(k + 1)

                    drain_write(nrt, nsemw)
                    fire(k + 1, nrt, nrc, nsemt, nsemc)
            else:
                @pl.when(k >= 1)
                def _():
                    drain_write(nrt, nsemw)

                fire(k + 1, nrt, nrc, nsemt, nsemc)

            def addrow(r, carry2):
                for j in range(_NVJ):
                    sl = pl.ds(j * _VEC, _VEC)
                    plsc.addupdate(rt.at[r, sl], rc[r, sl])
                return carry2

            lax.fori_loop(0, _C, addrow, 0)
            pltpu.async_copy(
                rt,
                out_hbm.at[pl.ds(pl.multiple_of(base_w + k * _C, _C), _C)],
                semw)
        return carry

    lax.fori_loop(0, n_chunks // 2, pair, 0)

    drain_write(rt0, sw0)
    drain_write(rt1, sw1)


def kernel(tokens, segment_ids, pos_ids, token_table, segment_table, pos_table):
    tok = jnp.reshape(tokens, (_N // _C, _C)).astype(jnp.int32)
    seg = jnp.reshape(segment_ids, (_N // _C, _C)).astype(jnp.int32)
    pos = jnp.reshape(pos_ids, (_N // _C, _C)).astype(jnp.int32)
    mesh = plsc.VectorSubcoreMesh(core_axis_name="c", subcore_axis_name="s")

    out = pl.kernel(
        _embed_body,
        mesh=mesh,
        out_type=jax.ShapeDtypeStruct((_N, _DIM), jnp.float32),
        scratch_types=[
            pltpu.VMEM((_KB, _C), jnp.int32),
            pltpu.VMEM((_KB, _C), jnp.int32),
            pltpu.VMEM((_KB, _C), jnp.int32),
            pltpu.VMEM((_KB, _C), jnp.int32),
            pltpu.VMEM((_C, _DIM), jnp.float32),
            pltpu.VMEM((_C, _DIM), jnp.float32),
            pltpu.VMEM((_C, _DIM), jnp.float32),
            pltpu.VMEM((_C, _DIM), jnp.float32),
            pltpu.VMEM((2, _DIM), jnp.float32),
            pltpu.VMEM((_CB_PER_T, _DIM), jnp.float32),
            pltpu.VMEM_SHARED((_COMB_ROWS, _DIM), jnp.float32),
            pltpu.SemaphoreType.DMA,
            pltpu.SemaphoreType.DMA,
            pltpu.SemaphoreType.DMA,
            pltpu.SemaphoreType.DMA,
            pltpu.SemaphoreType.DMA,
            pltpu.SemaphoreType.DMA,
        ],
    )(tok, seg, pos, token_table, segment_table, pos_table)
    return jnp.reshape(out, (_B, _L, _DIM))
